# Initial kernel scaffold; baseline (speedup 1.0000x reference)
import jax
import jax.numpy as jnp
from jax.experimental import pallas as pl


def kernel(hidden_states, W_sel):
    raise NotImplementedError("write your pallas kernel here")


# speedup vs baseline: 10.7945x; 10.7945x over previous
import functools

import jax
import jax.numpy as jnp
from jax.experimental import pallas as pl

def _affinity_body(w_ref, x_ref, out_ref):
    out_ref[...] = jax.lax.dot_general(
        w_ref[...], x_ref[...],
        (((1,), (1,)), ((), ())),
        preferred_element_type=jnp.float32,
    )


def _float_key(a):
    bits = jax.lax.bitcast_convert_type(a, jnp.int32)
    return jnp.where(bits >= 0, bits, bits ^ jnp.int32(0x7FFFFFFF))


def _stats_body(aff_ref, theta_ref, jcut_ref, mx_ref, denom_ref, *, C):
    a = aff_ref[...]
    E, T = a.shape
    key = _float_key(a)

    cnt_pos = jnp.sum((key >= 0).astype(jnp.int32), axis=1, keepdims=True)
    theta0 = jnp.where(cnt_pos >= C, 0, -2147483648).astype(jnp.int32)

    def _theta_step(i, prefix):
        cand = prefix + (jnp.int32(1) << (30 - i))
        cnt = jnp.sum((key >= cand).astype(jnp.int32), axis=1, keepdims=True)
        return jnp.where(cnt >= C, cand, prefix)

    theta = jax.lax.fori_loop(0, 31, _theta_step, theta0)

    gt = key > theta
    eq = key == theta
    n_gt = jnp.sum(gt.astype(jnp.int32), axis=1, keepdims=True)
    m = C - n_gt

    idx = jax.lax.broadcasted_iota(jnp.int32, (E, T), 1)

    def _j_step(i, J):
        cand = J + (jnp.int32(1) << (15 - i))
        cnt = jnp.sum((eq & (idx < cand)).astype(jnp.int32),
                      axis=1, keepdims=True)
        return jnp.where(cnt <= m, cand, J)

    jcut = jax.lax.fori_loop(0, 16, _j_step, jnp.zeros((E, 1), jnp.int32))

    sel = gt | (eq & (idx < jcut))
    mx = jnp.max(a, axis=1, keepdims=True)
    denom = jnp.sum(jnp.where(sel, jnp.exp(a - mx), 0.0),
                    axis=1, keepdims=True)

    theta_ref[...] = theta
    jcut_ref[...] = jcut
    mx_ref[...] = mx
    denom_ref[...] = denom


def _emit_body(aff_ref, theta_ref, jcut_ref, mx_ref, denom_ref,
               w_out_ref, a_out_ref, *, bt):
    i = pl.program_id(0)
    a = aff_ref[...]
    E = a.shape[0]
    key = _float_key(a)
    theta = theta_ref[...]
    idx = i * bt + jax.lax.broadcasted_iota(jnp.int32, (E, bt), 1)
    sel = (key > theta) | ((key == theta) & (idx < jcut_ref[...]))
    ex = jnp.exp(a - mx_ref[...])
    w_un = jnp.where(sel, ex / denom_ref[...], 0.0)
    cnt = jnp.sum(sel.astype(jnp.float32), axis=0, keepdims=True)
    w = w_un / jnp.maximum(cnt, 1.0)
    w_out_ref[...] = w.T
    a_out_ref[...] = sel.astype(jnp.float32).T


def kernel(hidden_states, W_sel):
    batch, seq, d_model = hidden_states.shape
    n_exp = W_sel.shape[0]
    num_tokens = batch * seq
    capacity = int(num_tokens * 1.0 / n_exp)
    C = min(capacity, num_tokens)

    x = hidden_states.reshape(num_tokens, d_model)

    bt = 2048
    aff_t = pl.pallas_call(
        _affinity_body,
        grid=(num_tokens // bt,),
        in_specs=[
            pl.BlockSpec((n_exp, d_model), lambda i: (0, 0)),
            pl.BlockSpec((bt, d_model), lambda i: (i, 0)),
        ],
        out_specs=pl.BlockSpec((n_exp, bt), lambda i: (0, i)),
        out_shape=jax.ShapeDtypeStruct((n_exp, num_tokens), jnp.float32),
    )(W_sel, x)

    stat_i32 = jax.ShapeDtypeStruct((n_exp, 1), jnp.int32)
    stat_f32 = jax.ShapeDtypeStruct((n_exp, 1), jnp.float32)
    theta, jcut, mx, denom = pl.pallas_call(
        functools.partial(_stats_body, C=C),
        out_shape=(stat_i32, stat_i32, stat_f32, stat_f32),
    )(aff_t)

    bt2 = 4096
    full_stat = pl.BlockSpec((n_exp, 1), lambda i: (0, 0))
    weights, assignments = pl.pallas_call(
        functools.partial(_emit_body, bt=bt2),
        grid=(num_tokens // bt2,),
        in_specs=[
            pl.BlockSpec((n_exp, bt2), lambda i: (0, i)),
            full_stat, full_stat, full_stat, full_stat,
        ],
        out_specs=(
            pl.BlockSpec((bt2, n_exp), lambda i: (i, 0)),
            pl.BlockSpec((bt2, n_exp), lambda i: (i, 0)),
        ),
        out_shape=(
            jax.ShapeDtypeStruct((num_tokens, n_exp), jnp.float32),
            jax.ShapeDtypeStruct((num_tokens, n_exp), jnp.float32),
        ),
    )(aff_t, theta, jcut, mx, denom)

    return weights, assignments, capacity


# speedup vs baseline: 11.7160x; 1.0854x over previous
import functools

import jax
import jax.numpy as jnp
from jax.experimental import pallas as pl

def _affinity_body(w_ref, x_ref, out_ref):
    out_ref[...] = jax.lax.dot_general(
        w_ref[...], x_ref[...],
        (((1,), (1,)), ((), ())),
        preferred_element_type=jnp.float32,
    )


def _float_key(a):
    bits = jax.lax.bitcast_convert_type(a, jnp.int32)
    return jnp.where(bits >= 0, bits, bits ^ jnp.int32(0x7FFFFFFF))


def _stats_body(aff_ref, theta_ref, jcut_ref, mx_ref, denom_ref, *, C):
    a = aff_ref[...]
    E, T = a.shape
    key = _float_key(a)

    cnt_pos = jnp.sum((key >= 0).astype(jnp.int32), axis=1, keepdims=True)
    theta0 = jnp.where(cnt_pos >= C, 0, -2147483648).astype(jnp.int32)

    def _count_ge(cand):
        return jnp.sum((key >= cand).astype(jnp.int32), axis=1, keepdims=True)

    def _theta_step2(i, prefix):
        hi = jnp.int32(1) << (30 - 2 * i)
        lo = jnp.int32(1) << (29 - 2 * i)
        c1 = prefix + lo
        c2 = prefix + hi
        c3 = prefix + hi + lo
        n1, n2, n3 = _count_ge(c1), _count_ge(c2), _count_ge(c3)
        return jnp.where(n3 >= C, c3,
                         jnp.where(n2 >= C, c2,
                                   jnp.where(n1 >= C, c1, prefix)))

    theta = jax.lax.fori_loop(0, 15, _theta_step2, theta0)
    cand = theta + 1
    theta = jnp.where(_count_ge(cand) >= C, cand, theta)

    gt = key > theta
    eq = key == theta
    n_gt = jnp.sum(gt.astype(jnp.int32), axis=1, keepdims=True)
    n_eq = jnp.sum(eq.astype(jnp.int32), axis=1, keepdims=True)
    m = C - n_gt

    def _j_fast(_):
        return jnp.full((E, 1), T, jnp.int32)

    idx = jax.lax.broadcasted_iota(jnp.int32, (E, T), 1)

    def _j_search(_):
        def _j_step(i, J):
            cand_j = J + (jnp.int32(1) << (15 - i))
            cnt = jnp.sum((eq & (idx < cand_j)).astype(jnp.int32),
                          axis=1, keepdims=True)
            return jnp.where(cnt <= m, cand_j, J)

        return jax.lax.fori_loop(0, 16, _j_step,
                                 jnp.zeros((E, 1), jnp.int32))

    jcut = jax.lax.cond(jnp.all(n_eq == m), _j_fast, _j_search, 0)

    sel = gt | (eq & (idx < jcut))
    mx = jnp.max(a, axis=1, keepdims=True)
    denom = jnp.sum(jnp.where(sel, jnp.exp(a - mx), 0.0),
                    axis=1, keepdims=True)

    theta_ref[...] = theta
    jcut_ref[...] = jcut
    mx_ref[...] = mx
    denom_ref[...] = denom


def _emit_body(aff_ref, theta_ref, jcut_ref, mx_ref, denom_ref,
               w_out_ref, a_out_ref, *, bt):
    i = pl.program_id(0)
    a = aff_ref[...]
    E = a.shape[0]
    key = _float_key(a)
    theta = theta_ref[...]
    idx = i * bt + jax.lax.broadcasted_iota(jnp.int32, (E, bt), 1)
    sel = (key > theta) | ((key == theta) & (idx < jcut_ref[...]))
    ex = jnp.exp(a - mx_ref[...])
    w_un = jnp.where(sel, ex / denom_ref[...], 0.0)
    cnt = jnp.sum(sel.astype(jnp.float32), axis=0, keepdims=True)
    w = w_un / jnp.maximum(cnt, 1.0)
    w_out_ref[...] = w.T
    a_out_ref[...] = sel.astype(jnp.float32).T


def kernel(hidden_states, W_sel):
    batch, seq, d_model = hidden_states.shape
    n_exp = W_sel.shape[0]
    num_tokens = batch * seq
    capacity = int(num_tokens * 1.0 / n_exp)
    C = min(capacity, num_tokens)

    x = hidden_states.reshape(num_tokens, d_model)

    bt = 4096
    aff_t = pl.pallas_call(
        _affinity_body,
        grid=(num_tokens // bt,),
        in_specs=[
            pl.BlockSpec((n_exp, d_model), lambda i: (0, 0)),
            pl.BlockSpec((bt, d_model), lambda i: (i, 0)),
        ],
        out_specs=pl.BlockSpec((n_exp, bt), lambda i: (0, i)),
        out_shape=jax.ShapeDtypeStruct((n_exp, num_tokens), jnp.float32),
    )(W_sel, x)

    stat_i32 = jax.ShapeDtypeStruct((n_exp, 1), jnp.int32)
    stat_f32 = jax.ShapeDtypeStruct((n_exp, 1), jnp.float32)
    theta, jcut, mx, denom = pl.pallas_call(
        functools.partial(_stats_body, C=C),
        out_shape=(stat_i32, stat_i32, stat_f32, stat_f32),
    )(aff_t)

    bt2 = 4096
    full_stat = pl.BlockSpec((n_exp, 1), lambda i: (0, 0))
    weights, assignments = pl.pallas_call(
        functools.partial(_emit_body, bt=bt2),
        grid=(num_tokens // bt2,),
        in_specs=[
            pl.BlockSpec((n_exp, bt2), lambda i: (0, i)),
            full_stat, full_stat, full_stat, full_stat,
        ],
        out_specs=(
            pl.BlockSpec((bt2, n_exp), lambda i: (i, 0)),
            pl.BlockSpec((bt2, n_exp), lambda i: (i, 0)),
        ),
        out_shape=(
            jax.ShapeDtypeStruct((num_tokens, n_exp), jnp.float32),
            jax.ShapeDtypeStruct((num_tokens, n_exp), jnp.float32),
        ),
    )(aff_t, theta, jcut, mx, denom)

    return weights, assignments, capacity
